# baseline (speedup 1.0000x reference)
import functools

import jax
import jax.numpy as jnp
from jax import lax
from jax.experimental import pallas as pl
from jax.experimental.pallas import tpu as pltpu
from jax.experimental.pallas import tpu_sc as plsc

_MIN_GAP = 2
_MAX_GAP = 16

_B = 4 * 3
_T = 32
_D = 224 * 224
_K = 8
_C = _D // _K
_FRAMES = _B * 2
_ITEMS = _FRAMES * _K


@functools.partial(
    pl.kernel,
    out_type=jax.ShapeDtypeStruct((_ITEMS, _C), jnp.float32),
    mesh=plsc.VectorSubcoreMesh(core_axis_name="c", subcore_axis_name="s"),
    scratch_types=[
        pltpu.VMEM((_K,), jnp.int32),
        pltpu.VMEM((_K, _C), jnp.float32),
        pltpu.SemaphoreType.DMA,
    ],
)
def _sc_gather(x_hbm, idx_hbm, out_hbm, idx_v, rows_v, sem):
    wid = lax.axis_index("s") * 2 + lax.axis_index("c")

    @pl.when(wid < _FRAMES)
    def _():
        pltpu.sync_copy(idx_hbm.at[pl.ds(wid * _K, _K)], idx_v)
        pltpu.async_copy(x_hbm.at[idx_v], rows_v, sem).wait()
        pltpu.sync_copy(rows_v, out_hbm.at[pl.ds(wid * _K, _K)])


def kernel(x):
    gap = jax.random.randint(
        jax.random.key(1), (1,), _MIN_GAP, _MAX_GAP).astype(jnp.int32)
    t_idx = jnp.concatenate([jnp.zeros((1,), dtype=jnp.int32), gap])
    base = jnp.arange(_B, dtype=jnp.int32) * _T
    src_rows = (base[:, None] + t_idx[None, :]).reshape(-1)
    chunk = jnp.arange(_K, dtype=jnp.int32)
    idx_arr = (src_rows[:, None] * _K + chunk[None, :]).reshape(-1)

    x_rows = x.reshape(_B * _T * _K, _C)
    out = _sc_gather(x_rows, idx_arr)
    return out.reshape(4, 3, 2, 224, 224)
